# baseline (speedup 1.0000x reference)
import functools

import jax
import jax.numpy as jnp
from jax import lax
from jax.experimental import pallas as pl
from jax.experimental.pallas import tpu as pltpu
from jax.experimental.pallas import tpu_sc as plsc

N_NODES = 50000
M = 16
IN_FEA = 128
EDGE_FEA = 16
AFL = 64
HID = 128
B = 500
NPC = 100

RB = 1000
GRID = N_NODES // RB
NE = N_NODES * M
MP = M // 2


_SC_CHUNK = 1000


def _sc_gather_body(ne, off, table_hbm, idx_hbm, out_hbm,
                    idx0, idx1, rows0, rows1, gsem, wsem0, wsem1):
    info = plsc.get_sparse_core_info()
    nw = info.num_cores * info.num_subcores
    b_per_w = ne // nw
    n_chunks = b_per_w // _SC_CHUNK
    wid = lax.axis_index("s") * info.num_cores + lax.axis_index("c")
    w_base = off + wid * b_per_w

    def fetch(i, idx_v, rows_v):
        base = w_base + i * _SC_CHUNK
        pltpu.sync_copy(idx_hbm.at[pl.ds(base, _SC_CHUNK)], idx_v)
        pltpu.async_copy(table_hbm.at[idx_v], rows_v, gsem).wait()

    def put(i, rows_v, wsem):
        base = wid * b_per_w + i * _SC_CHUNK
        return pltpu.async_copy(rows_v, out_hbm.at[pl.ds(base, _SC_CHUNK)], wsem)

    def drain(i, rows_v, wsem):
        base = wid * b_per_w + i * _SC_CHUNK
        pltpu.make_async_copy(rows_v, out_hbm.at[pl.ds(base, _SC_CHUNK)], wsem).wait()

    fetch(0, idx0, rows0)
    put(0, rows0, wsem0)
    fetch(1, idx1, rows1)
    put(1, rows1, wsem1)

    def pair(k, carry):
        a = 2 * k
        drain(a - 2, rows0, wsem0)
        fetch(a, idx0, rows0)
        put(a, rows0, wsem0)
        drain(a - 1, rows1, wsem1)
        fetch(a + 1, idx1, rows1)
        put(a + 1, rows1, wsem1)
        return carry

    lax.fori_loop(1, n_chunks // 2, pair, 0)

    if n_chunks % 2:
        last = n_chunks - 1
        drain(last - 2, rows0, wsem0)
        fetch(last, idx0, rows0)
        put(last, rows0, wsem0)
        drain(last, rows0, wsem0)
        drain(last - 1, rows1, wsem1)
    else:
        drain(n_chunks - 2, rows0, wsem0)
        drain(n_chunks - 1, rows1, wsem1)


def _sc_gather(table, idx, ne=NE, off=0):
    mesh = plsc.VectorSubcoreMesh(core_axis_name="c", subcore_axis_name="s")
    kern = functools.partial(
        pl.kernel,
        mesh=mesh,
        out_type=jax.ShapeDtypeStruct((ne, AFL), jnp.float32),
        scratch_types=[
            pltpu.VMEM((_SC_CHUNK,), jnp.int32),
            pltpu.VMEM((_SC_CHUNK,), jnp.int32),
            pltpu.VMEM((_SC_CHUNK, AFL), jnp.float32),
            pltpu.VMEM((_SC_CHUNK, AFL), jnp.float32),
            pltpu.SemaphoreType.DMA,
            pltpu.SemaphoreType.DMA,
            pltpu.SemaphoreType.DMA,
        ],
        compiler_params=pltpu.CompilerParams(use_tc_tiling_on_sc=False),
    )(functools.partial(_sc_gather_body, ne, off))
    return kern(table, idx)


def _full(shape):
    nd = len(shape)
    return pl.BlockSpec(shape, lambda i: (0,) * nd)


def _dot(a, b):
    return jnp.dot(a, b, preferred_element_type=jnp.float32)


def _embed_kernel(af_ref, we_ref, be_ref, wn_ref, x_ref, y_ref):
    x = _dot(af_ref[...], we_ref[...]) + be_ref[...]
    x_ref[...] = x
    y_ref[...] = _dot(x, wn_ref[...])


def _embed(atom_fea, we, be, wn0):
    return pl.pallas_call(
        _embed_kernel,
        grid=(GRID,),
        in_specs=[
            pl.BlockSpec((RB, IN_FEA), lambda i: (i, 0)),
            _full((IN_FEA, AFL)),
            _full((1, AFL)),
            _full((AFL, AFL)),
        ],
        out_specs=[
            pl.BlockSpec((RB, AFL), lambda i: (i, 0)),
            pl.BlockSpec((RB, AFL), lambda i: (i, 0)),
        ],
        out_shape=[
            jax.ShapeDtypeStruct((N_NODES, AFL), jnp.float32),
            jax.ShapeDtypeStruct((N_NODES, AFL), jnp.float32),
        ],
    )(atom_fea, we, be, wn0)


def _conv_core(x_ref, g_ref, nf_ref, w):
    xc = x_ref[...]
    pc2 = _dot(xc, w["wc2"][...]) + w["bc2"][...]
    s2 = jnp.zeros((RB, 2 * AFL), jnp.float32)
    for j in range(MP):
        g2 = g_ref[:, j, :] + w["bn2"][...]
        pe2 = _dot(nf_ref[:, 32 * j:32 * (j + 1)], w["bdwe"][...]) + w["be2"][...]
        i2 = pc2 * g2 * pe2
        zg = _dot(i2, w["wg2"][...]) + w["bg2"][...]
        zm = _dot(i2, w["wm2"][...]) + w["bm2"][...]
        s2 = s2 + jax.nn.sigmoid(zg) * jax.nn.softplus(zm)
    s = s2[:, :AFL] + s2[:, AFL:]
    mean = jnp.mean(s, axis=-1, keepdims=True)
    var = jnp.mean((s - mean) ** 2, axis=-1, keepdims=True)
    ln = (s - mean) * lax.rsqrt(var + 1e-6) * w["lns"][...] + w["lnb"][...]
    return xc + ln


_CONV_WNAMES = ["wc2", "bc2", "bn2", "bdwe", "be2", "wg2", "bg2", "wm2", "bm2",
                "lns", "lnb"]


def _conv_kernel(x_ref, g_ref, nf_ref, wc2, bc2, bn2, bdwe, be2, wg2, bg2,
                 wm2, bm2, lns, lnb, wnn, xo_ref, yo_ref):
    w = dict(zip(_CONV_WNAMES,
                 [wc2, bc2, bn2, bdwe, be2, wg2, bg2, wm2, bm2, lns, lnb]))
    xn = _conv_core(x_ref, g_ref, nf_ref, w)
    xo_ref[...] = xn
    yo_ref[...] = _dot(xn, wnn[...])


def _final_kernel(x_ref, g_ref, nf_ref, wc2, bc2, bn2, bdwe, be2, wg2, bg2,
                  wm2, bm2, lns, lnb, wh, bh, wo, bo, e_ref):
    w = dict(zip(_CONV_WNAMES,
                 [wc2, bc2, bn2, bdwe, be2, wg2, bg2, wm2, bm2, lns, lnb]))
    xn = _conv_core(x_ref, g_ref, nf_ref, w)
    h = jax.nn.softplus(_dot(xn, wh[...]) + bh[...])
    p = h * wo[...]
    nc = RB // NPC
    sel = (lax.broadcasted_iota(jnp.int32, (nc, RB), 1) // NPC
           == lax.broadcasted_iota(jnp.int32, (nc, RB), 0)).astype(jnp.float32)
    cs = _dot(sel, p)
    e = jnp.sum(cs, axis=1, keepdims=True) + NPC * bo[0, 0]
    e_ref[...] = jnp.broadcast_to(e, (nc, 128))[None]


_CONV_WSPECS = [
    _full((AFL, 2 * AFL)),
    _full((1, 2 * AFL)),
    _full((1, 2 * AFL)),
    _full((2 * EDGE_FEA, 2 * AFL)),
    _full((1, 2 * AFL)),
    _full((2 * AFL, 2 * AFL)),
    _full((1, 2 * AFL)),
    _full((2 * AFL, 2 * AFL)),
    _full((1, 2 * AFL)),
    _full((1, AFL)),
    _full((1, AFL)),
]

def _data_specs(off):
    return [
        pl.BlockSpec((RB, AFL), lambda i: (i, 0)),
        pl.BlockSpec((RB, MP, 2 * AFL), lambda i: (i, 0, 0)),
        pl.BlockSpec((RB, M * EDGE_FEA), lambda i, o=off: (i + o, 0)),
    ]


def _conv_layer(x, g3, nf2, cw, wn_next, rows, off):
    grid = rows // RB
    return pl.pallas_call(
        _conv_kernel,
        grid=(grid,),
        in_specs=_data_specs(off) + _CONV_WSPECS + [_full((AFL, AFL))],
        out_specs=[
            pl.BlockSpec((RB, AFL), lambda i: (i, 0)),
            pl.BlockSpec((RB, AFL), lambda i: (i, 0)),
        ],
        out_shape=[
            jax.ShapeDtypeStruct((rows, AFL), jnp.float32),
            jax.ShapeDtypeStruct((rows, AFL), jnp.float32),
        ],
    )(x, g3, nf2, *cw, wn_next)


def _final_layer(x, g3, nf2, cw, wh, bh, wo, bo, rows, off):
    nc = RB // NPC
    grid = rows // RB
    return pl.pallas_call(
        _final_kernel,
        grid=(grid,),
        in_specs=_data_specs(off) + _CONV_WSPECS + [
            _full((AFL, HID)),
            _full((1, HID)),
            _full((1, HID)),
            _full((1, 1)),
        ],
        out_specs=pl.BlockSpec((1, nc, 128), lambda i: (i, 0, 0)),
        out_shape=jax.ShapeDtypeStruct((grid, nc, 128), jnp.float32),
    )(x, g3, nf2, *cw, wh, bh, wo, bo)


def _conv_weights(c):
    z64 = jnp.zeros((AFL, AFL), jnp.float32)
    z32 = jnp.zeros((EDGE_FEA, AFL), jnp.float32)
    wg, wm = c["gate"]["W"], c["mag"]["W"]
    we = c["edge"]["W"]
    dup = lambda v: jnp.concatenate([v, v]).reshape(1, -1)
    return [
        jnp.concatenate([c["center"]["W"], c["center"]["W"]], axis=1),
        dup(c["center"]["b"]),
        dup(c["nbr"]["b"]),
        jnp.block([[we, z32], [z32, we]]),
        dup(c["edge"]["b"]),
        jnp.block([[wg, z64], [z64, wg]]),
        dup(c["gate"]["b"]),
        jnp.block([[wm, z64], [z64, wm]]),
        dup(c["mag"]["b"]),
        c["ln_scale"].reshape(1, -1),
        c["ln_bias"].reshape(1, -1),
    ]


_ROWS_A = 26000
_ROWS_B = N_NODES - _ROWS_A
_NE_A = _ROWS_A * M
_NE_B = _ROWS_B * M


def kernel(atom_fea, nbr_fea, nbr_fea_idx, params, batch_size, n_atoms_per_crystal):
    convs = params["convs"]
    idx = nbr_fea_idx.reshape(-1)
    nf2 = nbr_fea.reshape(N_NODES, M * EDGE_FEA)

    x, y = _embed(atom_fea, params["embed"]["W"],
                  params["embed"]["b"].reshape(1, -1), convs[0]["nbr"]["W"])
    xa, xb = x[:_ROWS_A], x[_ROWS_A:]

    hidw = (params["hid"]["W"], params["hid"]["b"].reshape(1, -1),
            params["out"]["W"].reshape(1, -1), params["out"]["b"].reshape(1, 1))

    for c in range(3):
        cw = _conv_weights(convs[c])
        ga = _sc_gather(y, idx, _NE_A, 0)
        gb = _sc_gather(y, idx, _NE_B, _NE_A)
        g3a = ga.reshape(_ROWS_A, MP, 2 * AFL)
        g3b = gb.reshape(_ROWS_B, MP, 2 * AFL)
        if c < 2:
            wnn = convs[c + 1]["nbr"]["W"]
            xa, ya = _conv_layer(xa, g3a, nf2, cw, wnn, _ROWS_A, 0)
            xb, yb = _conv_layer(xb, g3b, nf2, cw, wnn, _ROWS_B, _ROWS_A // RB)
            y = jnp.concatenate([ya, yb])
        else:
            ea = _final_layer(xa, g3a, nf2, cw, *hidw, _ROWS_A, 0)
            eb = _final_layer(xb, g3b, nf2, cw, *hidw, _ROWS_B, _ROWS_A // RB)
            e = jnp.concatenate([ea, eb]).reshape(B, 128)[:, :1]
    dep = (batch_size * n_atoms_per_crystal - B * NPC)
    return e + jnp.asarray(dep).astype(e.dtype)


# speedup vs baseline: 1.0207x; 1.0207x over previous
import functools

import jax
import jax.numpy as jnp
from jax import lax
from jax.experimental import pallas as pl
from jax.experimental.pallas import tpu as pltpu
from jax.experimental.pallas import tpu_sc as plsc

N_NODES = 50000
M = 16
IN_FEA = 128
EDGE_FEA = 16
AFL = 64
HID = 128
B = 500
NPC = 100

RB = 1000
GRID = N_NODES // RB
NE = N_NODES * M
MP = M // 2


_SC_CHUNK = 1000


def _sc_gather_body(ne, off, table_hbm, idx_hbm, out_hbm,
                    idx0, idx1, rows0, rows1, gsem, wsem0, wsem1):
    info = plsc.get_sparse_core_info()
    nw = info.num_cores * info.num_subcores
    b_per_w = ne // nw
    n_chunks = b_per_w // _SC_CHUNK
    wid = lax.axis_index("s") * info.num_cores + lax.axis_index("c")
    w_base = off + wid * b_per_w

    def fetch(i, idx_v, rows_v):
        base = w_base + i * _SC_CHUNK
        pltpu.sync_copy(idx_hbm.at[pl.ds(base, _SC_CHUNK)], idx_v)
        pltpu.async_copy(table_hbm.at[idx_v], rows_v, gsem).wait()

    def put(i, rows_v, wsem):
        base = wid * b_per_w + i * _SC_CHUNK
        return pltpu.async_copy(rows_v, out_hbm.at[pl.ds(base, _SC_CHUNK)], wsem)

    def drain(i, rows_v, wsem):
        base = wid * b_per_w + i * _SC_CHUNK
        pltpu.make_async_copy(rows_v, out_hbm.at[pl.ds(base, _SC_CHUNK)], wsem).wait()

    fetch(0, idx0, rows0)
    put(0, rows0, wsem0)
    fetch(1, idx1, rows1)
    put(1, rows1, wsem1)

    def pair(k, carry):
        a = 2 * k
        drain(a - 2, rows0, wsem0)
        fetch(a, idx0, rows0)
        put(a, rows0, wsem0)
        drain(a - 1, rows1, wsem1)
        fetch(a + 1, idx1, rows1)
        put(a + 1, rows1, wsem1)
        return carry

    lax.fori_loop(1, n_chunks // 2, pair, 0)

    if n_chunks % 2:
        last = n_chunks - 1
        drain(last - 2, rows0, wsem0)
        fetch(last, idx0, rows0)
        put(last, rows0, wsem0)
        drain(last, rows0, wsem0)
        drain(last - 1, rows1, wsem1)
    else:
        drain(n_chunks - 2, rows0, wsem0)
        drain(n_chunks - 1, rows1, wsem1)


def _sc_gather(table, idx, ne=NE, off=0):
    mesh = plsc.VectorSubcoreMesh(core_axis_name="c", subcore_axis_name="s")
    kern = functools.partial(
        pl.kernel,
        mesh=mesh,
        out_type=jax.ShapeDtypeStruct((ne, AFL), jnp.float32),
        scratch_types=[
            pltpu.VMEM((_SC_CHUNK,), jnp.int32),
            pltpu.VMEM((_SC_CHUNK,), jnp.int32),
            pltpu.VMEM((_SC_CHUNK, AFL), jnp.float32),
            pltpu.VMEM((_SC_CHUNK, AFL), jnp.float32),
            pltpu.SemaphoreType.DMA,
            pltpu.SemaphoreType.DMA,
            pltpu.SemaphoreType.DMA,
        ],
        compiler_params=pltpu.CompilerParams(use_tc_tiling_on_sc=False),
    )(functools.partial(_sc_gather_body, ne, off))
    return kern(table, idx)


def _full(shape):
    nd = len(shape)
    return pl.BlockSpec(shape, lambda i: (0,) * nd)


def _dot(a, b):
    return jnp.dot(a, b, preferred_element_type=jnp.float32)


def _embed_kernel(af_ref, we_ref, be_ref, wn_ref, x_ref, y_ref):
    x = _dot(af_ref[...], we_ref[...]) + be_ref[...]
    x_ref[...] = x
    y_ref[...] = _dot(x, wn_ref[...])


def _embed(atom_fea, we, be, wn0):
    return pl.pallas_call(
        _embed_kernel,
        grid=(GRID,),
        in_specs=[
            pl.BlockSpec((RB, IN_FEA), lambda i: (i, 0)),
            _full((IN_FEA, AFL)),
            _full((1, AFL)),
            _full((AFL, AFL)),
        ],
        out_specs=[
            pl.BlockSpec((RB, AFL), lambda i: (i, 0)),
            pl.BlockSpec((RB, AFL), lambda i: (i, 0)),
        ],
        out_shape=[
            jax.ShapeDtypeStruct((N_NODES, AFL), jnp.float32),
            jax.ShapeDtypeStruct((N_NODES, AFL), jnp.float32),
        ],
    )(atom_fea, we, be, wn0)


def _conv_core(x_ref, g_ref, nf_ref, w):
    xc = x_ref[...]
    pc2 = _dot(xc, w["wc2"][...]) + w["bc2"][...]
    s2 = jnp.zeros((RB, 2 * AFL), jnp.float32)
    for j in range(MP):
        g2 = g_ref[:, j, :] + w["bn2"][...]
        pe2 = _dot(nf_ref[:, 32 * j:32 * (j + 1)], w["bdwe"][...]) + w["be2"][...]
        i2 = pc2 * g2 * pe2
        z = _dot(i2, w["wgm2"][...]) + w["bgm2"][...]
        s2 = s2 + jax.nn.sigmoid(z[:, :128]) * jax.nn.softplus(z[:, 128:])
    s = s2[:, :AFL] + s2[:, AFL:]
    mean = jnp.mean(s, axis=-1, keepdims=True)
    var = jnp.mean((s - mean) ** 2, axis=-1, keepdims=True)
    ln = (s - mean) * lax.rsqrt(var + 1e-6) * w["lns"][...] + w["lnb"][...]
    return xc + ln


_CONV_WNAMES = ["wc2", "bc2", "bn2", "bdwe", "be2", "wgm2", "bgm2", "lns", "lnb"]


def _conv_kernel(x_ref, g_ref, nf_ref, wc2, bc2, bn2, bdwe, be2, wgm2, bgm2,
                 lns, lnb, wnn, xo_ref, yo_ref):
    w = dict(zip(_CONV_WNAMES, [wc2, bc2, bn2, bdwe, be2, wgm2, bgm2, lns, lnb]))
    xn = _conv_core(x_ref, g_ref, nf_ref, w)
    xo_ref[...] = xn
    yo_ref[...] = _dot(xn, wnn[...])


def _conv_kernel_alias(x_ref, g_ref, nf_ref, yin_ref, wc2, bc2, bn2, bdwe, be2,
                       wgm2, bgm2, lns, lnb, wnn, xo_ref, yo_ref):
    del yin_ref
    w = dict(zip(_CONV_WNAMES, [wc2, bc2, bn2, bdwe, be2, wgm2, bgm2, lns, lnb]))
    xn = _conv_core(x_ref, g_ref, nf_ref, w)
    xo_ref[...] = xn
    yo_ref[...] = _dot(xn, wnn[...])


def _final_kernel(x_ref, g_ref, nf_ref, wc2, bc2, bn2, bdwe, be2, wgm2, bgm2,
                  lns, lnb, wh, bh, wo, bo, e_ref):
    w = dict(zip(_CONV_WNAMES, [wc2, bc2, bn2, bdwe, be2, wgm2, bgm2, lns, lnb]))
    xn = _conv_core(x_ref, g_ref, nf_ref, w)
    h = jax.nn.softplus(_dot(xn, wh[...]) + bh[...])
    p = h * wo[...]
    nc = RB // NPC
    sel = (lax.broadcasted_iota(jnp.int32, (nc, RB), 1) // NPC
           == lax.broadcasted_iota(jnp.int32, (nc, RB), 0)).astype(jnp.float32)
    cs = _dot(sel, p)
    e = jnp.sum(cs, axis=1, keepdims=True) + NPC * bo[0, 0]
    e_ref[...] = jnp.broadcast_to(e, (nc, 128))[None]


_CONV_WSPECS = [
    _full((AFL, 2 * AFL)),
    _full((1, 2 * AFL)),
    _full((1, 2 * AFL)),
    _full((2 * EDGE_FEA, 2 * AFL)),
    _full((1, 2 * AFL)),
    _full((2 * AFL, 4 * AFL)),
    _full((1, 4 * AFL)),
    _full((1, AFL)),
    _full((1, AFL)),
]

def _data_specs(off):
    return [
        pl.BlockSpec((RB, AFL), lambda i: (i, 0)),
        pl.BlockSpec((RB, MP, 2 * AFL), lambda i: (i, 0, 0)),
        pl.BlockSpec((RB, M * EDGE_FEA), lambda i, o=off: (i + o, 0)),
    ]


def _conv_layer(x, g3, nf2, cw, wn_next, rows, off, y_in=None):
    grid = rows // RB
    y_spec = pl.BlockSpec((RB, AFL), lambda i, o=off: (i + o, 0))
    kern = _conv_kernel if y_in is None else _conv_kernel_alias
    extra_in = [] if y_in is None else [pl.BlockSpec(memory_space=pl.ANY)]
    extra_arg = [] if y_in is None else [y_in]
    return pl.pallas_call(
        kern,
        grid=(grid,),
        in_specs=_data_specs(off) + extra_in + _CONV_WSPECS + [_full((AFL, AFL))],
        out_specs=[
            pl.BlockSpec((RB, AFL), lambda i: (i, 0)),
            y_spec,
        ],
        out_shape=[
            jax.ShapeDtypeStruct((rows, AFL), jnp.float32),
            jax.ShapeDtypeStruct((N_NODES, AFL), jnp.float32),
        ],
        input_output_aliases={} if y_in is None else {3: 1},
    )(x, g3, nf2, *extra_arg, *cw, wn_next)


def _final_layer(x, g3, nf2, cw, wh, bh, wo, bo, rows, off):
    nc = RB // NPC
    grid = rows // RB
    return pl.pallas_call(
        _final_kernel,
        grid=(grid,),
        in_specs=_data_specs(off) + _CONV_WSPECS + [
            _full((AFL, HID)),
            _full((1, HID)),
            _full((1, HID)),
            _full((1, 1)),
        ],
        out_specs=pl.BlockSpec((1, nc, 128), lambda i: (i, 0, 0)),
        out_shape=jax.ShapeDtypeStruct((grid, nc, 128), jnp.float32),
    )(x, g3, nf2, *cw, wh, bh, wo, bo)


def _conv_weights(c):
    z64 = jnp.zeros((AFL, AFL), jnp.float32)
    z32 = jnp.zeros((EDGE_FEA, AFL), jnp.float32)
    wg, wm = c["gate"]["W"], c["mag"]["W"]
    we = c["edge"]["W"]
    dup = lambda v: jnp.concatenate([v, v]).reshape(1, -1)
    return [
        jnp.concatenate([c["center"]["W"], c["center"]["W"]], axis=1),
        dup(c["center"]["b"]),
        dup(c["nbr"]["b"]),
        jnp.block([[we, z32], [z32, we]]),
        dup(c["edge"]["b"]),
        jnp.block([[wg, z64, wm, z64], [z64, wg, z64, wm]]),
        jnp.concatenate([c["gate"]["b"], c["gate"]["b"],
                         c["mag"]["b"], c["mag"]["b"]]).reshape(1, -1),
        c["ln_scale"].reshape(1, -1),
        c["ln_bias"].reshape(1, -1),
    ]


_ROWS_A = 26000
_ROWS_B = N_NODES - _ROWS_A
_NE_A = _ROWS_A * M
_NE_B = _ROWS_B * M


def kernel(atom_fea, nbr_fea, nbr_fea_idx, params, batch_size, n_atoms_per_crystal):
    convs = params["convs"]
    idx = nbr_fea_idx.reshape(-1)
    nf2 = nbr_fea.reshape(N_NODES, M * EDGE_FEA)

    x, y = _embed(atom_fea, params["embed"]["W"],
                  params["embed"]["b"].reshape(1, -1), convs[0]["nbr"]["W"])
    xa, xb = x[:_ROWS_A], x[_ROWS_A:]

    hidw = (params["hid"]["W"], params["hid"]["b"].reshape(1, -1),
            params["out"]["W"].reshape(1, -1), params["out"]["b"].reshape(1, 1))

    for c in range(3):
        cw = _conv_weights(convs[c])
        ga = _sc_gather(y, idx, _NE_A, 0)
        gb = _sc_gather(y, idx, _NE_B, _NE_A)
        g3a = ga.reshape(_ROWS_A, MP, 2 * AFL)
        g3b = gb.reshape(_ROWS_B, MP, 2 * AFL)
        if c < 2:
            wnn = convs[c + 1]["nbr"]["W"]
            xa, ya = _conv_layer(xa, g3a, nf2, cw, wnn, _ROWS_A, 0)
            xb, y = _conv_layer(xb, g3b, nf2, cw, wnn, _ROWS_B, _ROWS_A // RB,
                                y_in=ya)
        else:
            ea = _final_layer(xa, g3a, nf2, cw, *hidw, _ROWS_A, 0)
            eb = _final_layer(xb, g3b, nf2, cw, *hidw, _ROWS_B, _ROWS_A // RB)
            e = jnp.concatenate([ea, eb]).reshape(B, 128)[:, :1]
    dep = (batch_size * n_atoms_per_crystal - B * NPC)
    return e + jnp.asarray(dep).astype(e.dtype)
